# baseline (speedup 1.0000x reference)
import functools

import jax
import jax.numpy as jnp
from jax import lax
from jax.experimental import pallas as pl
from jax.experimental.pallas import tpu as pltpu
from jax.experimental.pallas import tpu_sc as plsc

N_HEADS = 16
D_HEAD = 64
D_MODEL = N_HEADS * D_HEAD
SCALE = 0.125
N_Q = 32
NHQ = N_HEADS * N_Q
SLOTS = 32768
BUF = 16384

SC_CORES = 2
SC_SUBCORES = 16
N_WORKERS = SC_CORES * SC_SUBCORES

ROWS_PER_SUBCORE = BUF // SC_SUBCORES
GCHUNK = 32
N_GCHUNKS = ROWS_PER_SUBCORE // GCHUNK


def _gather_loop(cache_hbm, out_hbm, idx_v, base, bufs, gsems, wsems):
    gh = {}
    wh = {}

    def start_write(c):
        b = c % 3
        wh[c] = pltpu.async_copy(
            bufs[b], out_hbm.at[pl.ds(base + c * GCHUNK, GCHUNK)], wsems[b])

    for c in range(N_GCHUNKS):
        b = c % 3
        if c >= 3:
            wh[c - 3].wait()
        gh[c] = pltpu.async_copy(
            cache_hbm.at[idx_v.at[pl.ds(c * GCHUNK, GCHUNK)]],
            bufs[b], gsems[b])
        if c >= 1:
            gh[c - 1].wait()
            start_write(c - 1)
    gh[N_GCHUNKS - 1].wait()
    start_write(N_GCHUNKS - 1)
    for c in range(N_GCHUNKS - 3, N_GCHUNKS):
        wh[c].wait()


def _sc_gather_kernel(k_cache_hbm, v_cache_hbm, idx_hbm, k_out, v_out, idx_v,
                      r0, r1, r2, g0, g1, g2, w0, w1, w2):
    cid = lax.axis_index("c")
    sid = lax.axis_index("s")
    base = sid * ROWS_PER_SUBCORE
    pltpu.sync_copy(idx_hbm.at[pl.ds(base, ROWS_PER_SUBCORE)], idx_v)
    bufs = (r0, r1, r2)
    gsems = (g0, g1, g2)
    wsems = (w0, w1, w2)

    @pl.when(cid == 0)
    def _k():
        _gather_loop(k_cache_hbm, k_out, idx_v, base, bufs, gsems, wsems)

    @pl.when(cid == 1)
    def _v():
        _gather_loop(v_cache_hbm, v_out, idx_v, base, bufs, gsems, wsems)


def _sc_gather(kc2, vc2, ctx_idx):
    mesh = plsc.VectorSubcoreMesh(
        core_axis_name="c", subcore_axis_name="s",
        num_cores=SC_CORES, num_subcores=SC_SUBCORES)
    fn = pl.kernel(
        _sc_gather_kernel,
        out_type=(jax.ShapeDtypeStruct((BUF, D_MODEL), jnp.float32),
                  jax.ShapeDtypeStruct((BUF, D_MODEL), jnp.float32)),
        mesh=mesh,
        scratch_types=[
            pltpu.VMEM((ROWS_PER_SUBCORE,), jnp.int32),
            pltpu.VMEM((GCHUNK, D_MODEL), jnp.float32),
            pltpu.VMEM((GCHUNK, D_MODEL), jnp.float32),
            pltpu.VMEM((GCHUNK, D_MODEL), jnp.float32),
            pltpu.SemaphoreType.DMA,
            pltpu.SemaphoreType.DMA,
            pltpu.SemaphoreType.DMA,
            pltpu.SemaphoreType.DMA,
            pltpu.SemaphoreType.DMA,
            pltpu.SemaphoreType.DMA,
        ],
    )
    return fn(kc2, vc2, ctx_idx)


CH = 1024
N_CHUNKS = BUF // CH
BF = jnp.bfloat16
DN = (((1,), (1,)), ((), ()))
DS = (((1,), (0,)), ((), ()))


def _attn_kernel(qblk_ref, kb_ref, vb_ref, ctx_ref, alloc_ref, knew_ref,
                 vnew_ref, bias_ref, out_ref, m_ref, l_ref, acc_ref,
                 fix_ref):
    c = pl.program_id(0)

    @pl.when(c == 0)
    def _init():
        m_ref[...] = jnp.full_like(m_ref, -1e30)
        l_ref[...] = jnp.zeros_like(l_ref)
        acc_ref[...] = jnp.zeros_like(acc_ref)
        r = lax.broadcasted_iota(jnp.int32, (NHQ, N_Q), 0)
        j = lax.broadcasted_iota(jnp.int32, (NHQ, N_Q), 1)
        e = (lax.rem(r, N_Q) == j).astype(BF)
        snew = lax.dot_general(qblk_ref[...], knew_ref[...].astype(BF), DN,
                               preferred_element_type=jnp.float32)
        fix_ref[...] = jnp.concatenate([e, snew.astype(BF)], axis=1)

    ctxr = ctx_ref[0]
    best = jnp.full((1, CH), -1, jnp.int32)
    for j in range(N_Q):
        best = jnp.where(ctxr == alloc_ref[j], j, best)
    keep = (best < 0).astype(jnp.float32)
    onehot = (lax.broadcasted_iota(jnp.int32, (N_Q, CH), 0) == best
              ).astype(BF)

    kb = kb_ref[...].astype(BF)
    vb = vb_ref[...].astype(BF)

    s0 = lax.dot_general(qblk_ref[...], kb, DN,
                         preferred_element_type=jnp.float32)
    badd = jnp.concatenate([bias_ref[...].astype(BF), onehot], axis=0)
    s = s0 * keep + lax.dot_general(fix_ref[...], badd, DS,
                                    preferred_element_type=jnp.float32)

    m_old = m_ref[...]
    m_new = jnp.maximum(m_old, jnp.max(s, axis=1, keepdims=True))
    alpha = jnp.exp(m_old - m_new)
    p = jnp.exp(s - m_new)
    l_ref[...] = alpha * l_ref[...] + jnp.sum(p, axis=1, keepdims=True)
    m_ref[...] = m_new

    pk = (p * keep).astype(BF)
    pnew = lax.dot_general(p.astype(BF), onehot, DN,
                           preferred_element_type=jnp.float32)
    pv = (lax.dot_general(pk, vb, DS,
                          preferred_element_type=jnp.float32)
          + lax.dot_general(pnew.astype(BF), vnew_ref[...].astype(BF), DS,
                            preferred_element_type=jnp.float32))

    for h in range(N_HEADS):
        rs = slice(h * N_Q, (h + 1) * N_Q)
        cs = slice(h * D_HEAD, (h + 1) * D_HEAD)
        acc_ref[rs, :] = alpha[rs] * acc_ref[rs, :] + pv[rs, cs]

    @pl.when(c == N_CHUNKS - 1)
    def _fin():
        for h in range(N_HEADS):
            rs = slice(h * N_Q, (h + 1) * N_Q)
            cs = slice(h * D_HEAD, (h + 1) * D_HEAD)
            out_ref[:, cs] = acc_ref[rs, :] / l_ref[rs]


def _tc_attention(qblk, k_buf, v_buf, ctx_r, alloc, knew, vnew, attn_bias):
    return pl.pallas_call(
        _attn_kernel,
        grid=(N_CHUNKS,),
        in_specs=[
            pl.BlockSpec((NHQ, D_MODEL), lambda c: (0, 0)),
            pl.BlockSpec((CH, D_MODEL), lambda c: (c, 0)),
            pl.BlockSpec((CH, D_MODEL), lambda c: (c, 0)),
            pl.BlockSpec((1, 1, CH), lambda c: (c, 0, 0)),
            pl.BlockSpec(memory_space=pltpu.SMEM),
            pl.BlockSpec((N_Q, D_MODEL), lambda c: (0, 0)),
            pl.BlockSpec((N_Q, D_MODEL), lambda c: (0, 0)),
            pl.BlockSpec((N_Q, CH), lambda c: (0, c)),
        ],
        out_specs=pl.BlockSpec((N_Q, D_MODEL), lambda c: (0, 0)),
        out_shape=jax.ShapeDtypeStruct((N_Q, D_MODEL), jnp.float32),
        scratch_shapes=[
            pltpu.VMEM((NHQ, 1), jnp.float32),
            pltpu.VMEM((NHQ, 1), jnp.float32),
            pltpu.VMEM((NHQ, D_HEAD), jnp.float32),
            pltpu.VMEM((NHQ, 2 * N_Q), BF),
        ],
    )(qblk, k_buf, v_buf, ctx_r, alloc, knew, vnew, attn_bias)


def _build_qblk(q):
    qt = jnp.transpose(q, (1, 0, 2)) * SCALE
    eye = jnp.eye(N_HEADS, dtype=q.dtype)
    qblk = jnp.einsum('hqd,hg->hqgd', qt, eye)
    return qblk.reshape(NHQ, D_MODEL).astype(BF)


def kernel(q, k, v, k_cache, v_cache, allocated_index_tensor,
           context_index_tensor, attn_bias):
    ctx = context_index_tensor.astype(jnp.int32)
    alloc = allocated_index_tensor.astype(jnp.int32)
    k_buf, v_buf = _sc_gather(
        k_cache.reshape(SLOTS, D_MODEL), v_cache.reshape(SLOTS, D_MODEL), ctx)
    out = _tc_attention(
        _build_qblk(q), k_buf, v_buf,
        ctx.reshape(N_CHUNKS, 1, CH), alloc,
        k.reshape(N_Q, D_MODEL), v.reshape(N_Q, D_MODEL), attn_bias)
    return out


# speedup vs baseline: 1.0271x; 1.0271x over previous
import functools

import jax
import jax.numpy as jnp
from jax import lax
from jax.experimental import pallas as pl
from jax.experimental.pallas import tpu as pltpu
from jax.experimental.pallas import tpu_sc as plsc

N_HEADS = 16
D_HEAD = 64
D_MODEL = N_HEADS * D_HEAD
SCALE = 0.125
N_Q = 32
NHQ = N_HEADS * N_Q
SLOTS = 32768
BUF = 16384

SC_CORES = 2
SC_SUBCORES = 16
N_WORKERS = SC_CORES * SC_SUBCORES

ROWS_PER_WORKER = BUF // N_WORKERS
GCHUNK = 32
N_GCHUNKS = ROWS_PER_WORKER // GCHUNK


def _sc_gather_kernel(cache_hbm, idx_hbm, out_hbm, idx_v,
                      r0, r1, r2, g0, g1, g2, w0, w1, w2):
    cid = lax.axis_index("c")
    sid = lax.axis_index("s")
    wid = sid * SC_CORES + cid
    base = wid * ROWS_PER_WORKER

    pltpu.sync_copy(idx_hbm.at[pl.ds(base, ROWS_PER_WORKER)], idx_v)
    bufs = (r0, r1, r2)
    gsems = (g0, g1, g2)
    wsems = (w0, w1, w2)
    gh = {}
    wh = {}

    def start_write(c):
        b = c % 3
        wh[c] = pltpu.async_copy(
            bufs[b], out_hbm.at[pl.ds(base + c * GCHUNK, GCHUNK)], wsems[b])

    for c in range(N_GCHUNKS):
        b = c % 3
        if c >= 3:
            wh[c - 3].wait()
        gh[c] = pltpu.async_copy(
            cache_hbm.at[idx_v.at[pl.ds(c * GCHUNK, GCHUNK)]],
            bufs[b], gsems[b])
        if c >= 1:
            gh[c - 1].wait()
            start_write(c - 1)
    gh[N_GCHUNKS - 1].wait()
    start_write(N_GCHUNKS - 1)
    for c in range(N_GCHUNKS - 3, N_GCHUNKS):
        wh[c].wait()


def _sc_gather(cache2d, ctx_idx):
    mesh = plsc.VectorSubcoreMesh(
        core_axis_name="c", subcore_axis_name="s",
        num_cores=SC_CORES, num_subcores=SC_SUBCORES)
    fn = pl.kernel(
        _sc_gather_kernel,
        out_type=jax.ShapeDtypeStruct((BUF, D_MODEL), jnp.float32),
        mesh=mesh,
        scratch_types=[
            pltpu.VMEM((ROWS_PER_WORKER,), jnp.int32),
            pltpu.VMEM((GCHUNK, D_MODEL), jnp.float32),
            pltpu.VMEM((GCHUNK, D_MODEL), jnp.float32),
            pltpu.VMEM((GCHUNK, D_MODEL), jnp.float32),
            pltpu.SemaphoreType.DMA,
            pltpu.SemaphoreType.DMA,
            pltpu.SemaphoreType.DMA,
            pltpu.SemaphoreType.DMA,
            pltpu.SemaphoreType.DMA,
            pltpu.SemaphoreType.DMA,
        ],
    )
    return fn(cache2d, ctx_idx)


CH = 1024
N_CHUNKS = BUF // CH
BF = jnp.bfloat16
DN = (((1,), (1,)), ((), ()))
DS = (((1,), (0,)), ((), ()))


def _attn_kernel(qblk_ref, kb_ref, vb_ref, ctx_ref, alloc_ref, knew_ref,
                 vnew_ref, bias_ref, out_ref, m_ref, l_ref, acc_ref,
                 fix_ref):
    c = pl.program_id(0)

    @pl.when(c == 0)
    def _init():
        m_ref[...] = jnp.full_like(m_ref, -1e30)
        l_ref[...] = jnp.zeros_like(l_ref)
        acc_ref[...] = jnp.zeros_like(acc_ref)
        r = lax.broadcasted_iota(jnp.int32, (NHQ, N_Q), 0)
        j = lax.broadcasted_iota(jnp.int32, (NHQ, N_Q), 1)
        e = (lax.rem(r, N_Q) == j).astype(BF)
        snew = lax.dot_general(qblk_ref[...], knew_ref[...].astype(BF), DN,
                               preferred_element_type=jnp.float32)
        fix_ref[...] = jnp.concatenate([e, snew.astype(BF)], axis=1)

    ctxr = ctx_ref[0]
    best = jnp.full((1, CH), -1, jnp.int32)
    for j in range(N_Q):
        best = jnp.where(ctxr == alloc_ref[j], j, best)
    keep = (best < 0).astype(jnp.float32)
    onehot = (lax.broadcasted_iota(jnp.int32, (N_Q, CH), 0) == best
              ).astype(BF)

    kb = kb_ref[...].astype(BF)
    vb = vb_ref[...].astype(BF)

    s0 = lax.dot_general(qblk_ref[...], kb, DN,
                         preferred_element_type=jnp.float32)
    badd = jnp.concatenate([bias_ref[...].astype(BF), onehot], axis=0)
    s = s0 * keep + lax.dot_general(fix_ref[...], badd, DS,
                                    preferred_element_type=jnp.float32)

    m_old = m_ref[...]
    m_new = jnp.maximum(m_old, jnp.max(s, axis=1, keepdims=True))
    alpha = jnp.exp(m_old - m_new)
    p = jnp.exp(s - m_new)
    l_ref[...] = alpha * l_ref[...] + jnp.sum(p, axis=1, keepdims=True)
    m_ref[...] = m_new

    pk = (p * keep).astype(BF)
    pnew = lax.dot_general(p.astype(BF), onehot, DN,
                           preferred_element_type=jnp.float32)
    pv = (lax.dot_general(pk, vb, DS,
                          preferred_element_type=jnp.float32)
          + lax.dot_general(pnew.astype(BF), vnew_ref[...].astype(BF), DS,
                            preferred_element_type=jnp.float32))

    for h in range(N_HEADS):
        rs = slice(h * N_Q, (h + 1) * N_Q)
        cs = slice(h * D_HEAD, (h + 1) * D_HEAD)
        acc_ref[rs, :] = alpha[rs] * acc_ref[rs, :] + pv[rs, cs]

    @pl.when(c == N_CHUNKS - 1)
    def _fin():
        for h in range(N_HEADS):
            rs = slice(h * N_Q, (h + 1) * N_Q)
            cs = slice(h * D_HEAD, (h + 1) * D_HEAD)
            out_ref[:, cs] = acc_ref[rs, :] / l_ref[rs]


def _tc_attention(qblk, k_buf, v_buf, ctx_r, alloc, knew, vnew, attn_bias):
    return pl.pallas_call(
        _attn_kernel,
        grid=(N_CHUNKS,),
        in_specs=[
            pl.BlockSpec((NHQ, D_MODEL), lambda c: (0, 0)),
            pl.BlockSpec((CH, D_MODEL), lambda c: (c, 0)),
            pl.BlockSpec((CH, D_MODEL), lambda c: (c, 0)),
            pl.BlockSpec((1, 1, CH), lambda c: (c, 0, 0)),
            pl.BlockSpec(memory_space=pltpu.SMEM),
            pl.BlockSpec((N_Q, D_MODEL), lambda c: (0, 0)),
            pl.BlockSpec((N_Q, D_MODEL), lambda c: (0, 0)),
            pl.BlockSpec((N_Q, CH), lambda c: (0, c)),
        ],
        out_specs=pl.BlockSpec((N_Q, D_MODEL), lambda c: (0, 0)),
        out_shape=jax.ShapeDtypeStruct((N_Q, D_MODEL), jnp.float32),
        scratch_shapes=[
            pltpu.VMEM((NHQ, 1), jnp.float32),
            pltpu.VMEM((NHQ, 1), jnp.float32),
            pltpu.VMEM((NHQ, D_HEAD), jnp.float32),
            pltpu.VMEM((NHQ, 2 * N_Q), BF),
        ],
    )(qblk, k_buf, v_buf, ctx_r, alloc, knew, vnew, attn_bias)


def _build_qblk(q):
    qt = jnp.transpose(q, (1, 0, 2)) * SCALE
    eye = jnp.eye(N_HEADS, dtype=q.dtype)
    qblk = jnp.einsum('hqd,hg->hqgd', qt, eye)
    return qblk.reshape(NHQ, D_MODEL).astype(BF)


def kernel(q, k, v, k_cache, v_cache, allocated_index_tensor,
           context_index_tensor, attn_bias):
    ctx = context_index_tensor.astype(jnp.int32)
    alloc = allocated_index_tensor.astype(jnp.int32)
    k_buf = _sc_gather(k_cache.reshape(SLOTS, D_MODEL), ctx)
    v_buf = _sc_gather(v_cache.reshape(SLOTS, D_MODEL), ctx)
    out = _tc_attention(
        _build_qblk(q), k_buf, v_buf,
        ctx.reshape(N_CHUNKS, 1, CH), alloc,
        k.reshape(N_Q, D_MODEL), v.reshape(N_Q, D_MODEL), attn_bias)
    return out
